# baseline (speedup 1.0000x reference)
import functools

import jax
import jax.numpy as jnp
from jax import lax
from jax.experimental import pallas as pl
from jax.experimental.pallas import tpu as pltpu
from jax.experimental.pallas import tpu_sc as plsc

_NC = 2
_NS = 16
_NW = _NC * _NS
_CHUNK = 128
_DEGW = 16
_FRAC0 = 0.8


def _ceil_div(a, b):
    return -(-a // b)


def _sc_mesh():
    return plsc.VectorSubcoreMesh(core_axis_name="c", subcore_axis_name="s")


def _make_deg_kernel(n_acc, cpw):
    rows_per_tile = n_acc // _NS
    zc = rows_per_tile // _CHUNK

    @functools.partial(
        pl.kernel,
        mesh=_sc_mesh(),
        out_type=(
            jax.ShapeDtypeStruct((n_acc, _DEGW), jnp.float32),
            jax.ShapeDtypeStruct((n_acc, _DEGW), jnp.float32),
        ),
        scratch_types=[
            pltpu.VMEM((_CHUNK,), jnp.int32),
            pltpu.VMEM((_CHUNK, _DEGW), jnp.float32),
            pltpu.VMEM_SHARED((n_acc, _DEGW), jnp.float32),
        ],
    )
    def deg_kernel(dst_hbm, d0_hbm, d1_hbm, idx_v, ones_v, acc):
        cid = lax.axis_index("c")
        sid = lax.axis_index("s")
        wid = cid * _NS + sid

        def fill(val):
            def body(r, carry):
                ones_v[r, pl.ds(0, _DEGW)] = jnp.full((_DEGW,), val, jnp.float32)
                return carry

            lax.fori_loop(0, _CHUNK, body, 0)

        fill(0.0)
        zrow0 = sid * rows_per_tile
        for z in range(zc):
            pltpu.sync_copy(ones_v, acc.at[pl.ds(zrow0 + z * _CHUNK, _CHUNK)])
        fill(1.0)
        plsc.subcore_barrier()

        def edge_body(g, carry):
            base = (wid * cpw + g) * _CHUNK
            pltpu.sync_copy(dst_hbm.at[pl.ds(base, _CHUNK)], idx_v)
            pltpu.sync_copy(ones_v, acc.at[idx_v], add=True)
            return carry

        lax.fori_loop(0, cpw, edge_body, 0)
        plsc.subcore_barrier()

        @pl.when(cid == 0)
        def _():
            pltpu.sync_copy(
                acc.at[pl.ds(zrow0, rows_per_tile)],
                d0_hbm.at[pl.ds(zrow0, rows_per_tile)],
            )

        @pl.when(cid == 1)
        def _():
            pltpu.sync_copy(
                acc.at[pl.ds(zrow0, rows_per_tile)],
                d1_hbm.at[pl.ds(zrow0, rows_per_tile)],
            )

    return deg_kernel


def _make_agg_kernel(n, d, n_acc, cpw0, cpw1):
    rows_per_tile = n_acc // _NS
    zc = rows_per_tile // _CHUNK

    @functools.partial(
        pl.kernel,
        mesh=_sc_mesh(),
        out_type=(
            jax.ShapeDtypeStruct((n_acc, d), jnp.float32),
            jax.ShapeDtypeStruct((n_acc, d), jnp.float32),
        ),
        scratch_types=[
            pltpu.VMEM((_CHUNK,), jnp.int32),
            pltpu.VMEM((_CHUNK,), jnp.int32),
            pltpu.VMEM((_CHUNK, d), jnp.float32),
            pltpu.VMEM_SHARED((n_acc, d), jnp.float32),
            pltpu.SemaphoreType.DMA,
        ],
    )
    def agg_kernel(hs_hbm, src_hbm, dst_hbm, p0_hbm, p1_hbm, src_v, dst_v, rows_v, acc, sem):
        cid = lax.axis_index("c")
        sid = lax.axis_index("s")

        def zero_body(r, carry):
            for c in range(d // 16):
                rows_v[r, pl.ds(c * 16, 16)] = jnp.zeros((16,), jnp.float32)
            return carry

        lax.fori_loop(0, _CHUNK, zero_body, 0)
        zrow0 = sid * rows_per_tile
        for z in range(zc):
            pltpu.sync_copy(rows_v, acc.at[pl.ds(zrow0 + z * _CHUNK, _CHUNK)])
        plsc.subcore_barrier()

        def make_edge_body(start):
            def edge_body(g, carry):
                base = (start + g) * _CHUNK
                pltpu.sync_copy(src_hbm.at[pl.ds(base, _CHUNK)], src_v)
                pltpu.sync_copy(dst_hbm.at[pl.ds(base, _CHUNK)], dst_v)
                pltpu.async_copy(hs_hbm.at[src_v], rows_v, sem).wait()
                pltpu.sync_copy(rows_v, acc.at[dst_v], add=True)
                return carry

            return edge_body

        @pl.when(cid == 0)
        def _():
            lax.fori_loop(0, cpw0, make_edge_body(sid * cpw0), 0)

        @pl.when(cid == 1)
        def _():
            lax.fori_loop(0, cpw1, make_edge_body(_NS * cpw0 + sid * cpw1), 0)

        plsc.subcore_barrier()

        @pl.when(cid == 0)
        def _():
            pltpu.sync_copy(
                acc.at[pl.ds(zrow0, rows_per_tile)],
                p0_hbm.at[pl.ds(zrow0, rows_per_tile)],
            )

        @pl.when(cid == 1)
        def _():
            pltpu.sync_copy(
                acc.at[pl.ds(zrow0, rows_per_tile)],
                p1_hbm.at[pl.ds(zrow0, rows_per_tile)],
            )

    return agg_kernel


def _pick_bn(n):
    for bn in (1024, 1000, 512, 500, 256, 250, 128, 8):
        if n % bn == 0:
            return bn
    return n


def _tc_hs(x, w, d0, d1):
    n, d = x.shape
    bn = _pick_bn(n)

    def body(x_ref, w_ref, d0_ref, d1_ref, o_ref):
        deg = d0_ref[...][:, :1] + d1_ref[...][:, :1] + 1.0
        dinv = lax.rsqrt(deg)
        o_ref[...] = (
            jnp.dot(x_ref[...], w_ref[...], preferred_element_type=jnp.float32) * dinv
        )

    return pl.pallas_call(
        body,
        grid=(n // bn,),
        in_specs=[
            pl.BlockSpec((bn, d), lambda i: (i, 0)),
            pl.BlockSpec((d, d), lambda i: (0, 0)),
            pl.BlockSpec((bn, _DEGW), lambda i: (i, 0)),
            pl.BlockSpec((bn, _DEGW), lambda i: (i, 0)),
        ],
        out_specs=pl.BlockSpec((bn, d), lambda i: (i, 0)),
        out_shape=jax.ShapeDtypeStruct((n, d), jnp.float32),
    )(x, w, d0, d1)


def _tc_mid(p0, p1, hs1, d0, d1, b1, w2):
    n, d = hs1.shape
    bn = _pick_bn(n)

    def body(p0_ref, p1_ref, hs1_ref, d0_ref, d1_ref, b1_ref, w2_ref, t1_ref, hs2_ref):
        deg = d0_ref[...][:, :1] + d1_ref[...][:, :1] + 1.0
        dinv = lax.rsqrt(deg)
        t1 = jnp.tanh((p0_ref[...] + p1_ref[...] + hs1_ref[...]) * dinv + b1_ref[...])
        t1_ref[...] = t1
        hs2_ref[...] = (
            jnp.dot(t1, w2_ref[...], preferred_element_type=jnp.float32) * dinv
        )

    return pl.pallas_call(
        body,
        grid=(n // bn,),
        in_specs=[
            pl.BlockSpec((bn, d), lambda i: (i, 0)),
            pl.BlockSpec((bn, d), lambda i: (i, 0)),
            pl.BlockSpec((bn, d), lambda i: (i, 0)),
            pl.BlockSpec((bn, _DEGW), lambda i: (i, 0)),
            pl.BlockSpec((bn, _DEGW), lambda i: (i, 0)),
            pl.BlockSpec((1, d), lambda i: (0, 0)),
            pl.BlockSpec((d, d), lambda i: (0, 0)),
        ],
        out_specs=[
            pl.BlockSpec((bn, d), lambda i: (i, 0)),
            pl.BlockSpec((bn, d), lambda i: (i, 0)),
        ],
        out_shape=[
            jax.ShapeDtypeStruct((n, d), jnp.float32),
            jax.ShapeDtypeStruct((n, d), jnp.float32),
        ],
    )(p0, p1, hs1, d0, d1, b1, w2)


def _tc_fin(q0, q1, hs2, d0, d1, b2, t1):
    n, d = hs2.shape
    bn = _pick_bn(n)

    def body(q0_ref, q1_ref, hs2_ref, d0_ref, d1_ref, b2_ref, t1_ref, o_ref):
        deg = d0_ref[...][:, :1] + d1_ref[...][:, :1] + 1.0
        dinv = lax.rsqrt(deg)
        t2 = jnp.tanh((q0_ref[...] + q1_ref[...] + hs2_ref[...]) * dinv + b2_ref[...])
        o_ref[:, 0, :] = t1_ref[...]
        o_ref[:, 1, :] = t2

    return pl.pallas_call(
        body,
        grid=(n // bn,),
        in_specs=[
            pl.BlockSpec((bn, d), lambda i: (i, 0)),
            pl.BlockSpec((bn, d), lambda i: (i, 0)),
            pl.BlockSpec((bn, d), lambda i: (i, 0)),
            pl.BlockSpec((bn, _DEGW), lambda i: (i, 0)),
            pl.BlockSpec((bn, _DEGW), lambda i: (i, 0)),
            pl.BlockSpec((1, d), lambda i: (0, 0)),
            pl.BlockSpec((bn, d), lambda i: (i, 0)),
        ],
        out_specs=pl.BlockSpec((bn, 2, d), lambda i: (i, 0, 0)),
        out_shape=jax.ShapeDtypeStruct((n, 2, d), jnp.float32),
    )(q0, q1, hs2, d0, d1, b2, t1)


def kernel(x, edge_index, W1, b1, W2, b2):
    n, d = x.shape
    e = edge_index.shape[1]
    s_pair = 2 * _ceil_div(_ceil_div(e, _CHUNK), 2 * _NS)
    cpw0 = max(1, min(s_pair - 1, round(s_pair * _FRAC0)))
    cpw0 = 2
    cpw1 = s_pair - cpw0
    cpw = s_pair // 2
    e_pad = s_pair * _NS * _CHUNK
    pad = e_pad - e
    src = jnp.concatenate([edge_index[0], jnp.zeros((pad,), edge_index.dtype)])
    dst = jnp.concatenate([edge_index[1], jnp.full((pad,), n, edge_index.dtype)])
    n_acc = _ceil_div(n + 1, _NS * _CHUNK) * _NS * _CHUNK

    d0, d1 = _make_deg_kernel(n_acc, cpw)(dst)
    hs1 = _tc_hs(x, W1, d0, d1)
    agg = _make_agg_kernel(n, d, n_acc, cpw0, cpw1)
    p0, p1 = agg(hs1, src, dst)
    t1, hs2 = _tc_mid(p0, p1, hs1, d0, d1, b1.reshape(1, d), W2)
    q0, q1 = agg(hs2, src, dst)
    return _tc_fin(q0, q1, hs2, d0, d1, b2.reshape(1, d), t1)


# speedup vs baseline: 1.6557x; 1.6557x over previous
import functools

import jax
import jax.numpy as jnp
from jax import lax
from jax.experimental import pallas as pl
from jax.experimental.pallas import tpu as pltpu
from jax.experimental.pallas import tpu_sc as plsc

_NC = 2
_NS = 16
_NW = _NC * _NS
_CHUNK = 128
_DEGW = 16
_FRAC0 = 0.65


def _ceil_div(a, b):
    return -(-a // b)


def _sc_mesh():
    return plsc.VectorSubcoreMesh(core_axis_name="c", subcore_axis_name="s")


def _make_deg_kernel(n_acc, cpw):
    rows_per_tile = n_acc // _NS
    zc = rows_per_tile // _CHUNK

    @functools.partial(
        pl.kernel,
        mesh=_sc_mesh(),
        out_type=(
            jax.ShapeDtypeStruct((n_acc, _DEGW), jnp.float32),
            jax.ShapeDtypeStruct((n_acc, _DEGW), jnp.float32),
        ),
        scratch_types=[
            pltpu.VMEM((_CHUNK,), jnp.int32),
            pltpu.VMEM((_CHUNK, _DEGW), jnp.float32),
            pltpu.VMEM_SHARED((n_acc, _DEGW), jnp.float32),
        ],
    )
    def deg_kernel(dst_hbm, d0_hbm, d1_hbm, idx_v, ones_v, acc):
        cid = lax.axis_index("c")
        sid = lax.axis_index("s")
        wid = cid * _NS + sid

        def fill(val):
            def body(r, carry):
                ones_v[r, pl.ds(0, _DEGW)] = jnp.full((_DEGW,), val, jnp.float32)
                return carry

            lax.fori_loop(0, _CHUNK, body, 0)

        fill(0.0)
        zrow0 = sid * rows_per_tile
        for z in range(zc):
            pltpu.sync_copy(ones_v, acc.at[pl.ds(zrow0 + z * _CHUNK, _CHUNK)])
        fill(1.0)
        plsc.subcore_barrier()

        def edge_body(g, carry):
            base = (wid * cpw + g) * _CHUNK
            pltpu.sync_copy(dst_hbm.at[pl.ds(base, _CHUNK)], idx_v)
            pltpu.sync_copy(ones_v, acc.at[idx_v], add=True)
            return carry

        lax.fori_loop(0, cpw, edge_body, 0)
        plsc.subcore_barrier()

        @pl.when(cid == 0)
        def _():
            pltpu.sync_copy(
                acc.at[pl.ds(zrow0, rows_per_tile)],
                d0_hbm.at[pl.ds(zrow0, rows_per_tile)],
            )

        @pl.when(cid == 1)
        def _():
            pltpu.sync_copy(
                acc.at[pl.ds(zrow0, rows_per_tile)],
                d1_hbm.at[pl.ds(zrow0, rows_per_tile)],
            )

    return deg_kernel


def _make_agg_kernel(n, d, n_acc, cpw0, cpw1):
    rows_per_tile = n_acc // _NS
    zc = rows_per_tile // _CHUNK

    @functools.partial(
        pl.kernel,
        mesh=_sc_mesh(),
        out_type=(
            jax.ShapeDtypeStruct((n_acc, d), jnp.float32),
            jax.ShapeDtypeStruct((n_acc, d), jnp.float32),
        ),
        scratch_types=[
            pltpu.VMEM((_CHUNK,), jnp.int32),
            pltpu.VMEM((_CHUNK,), jnp.int32),
            pltpu.VMEM((_CHUNK, d), jnp.float32),
            pltpu.VMEM_SHARED((n_acc, d), jnp.float32),
            pltpu.SemaphoreType.DMA,
        ],
    )
    def agg_kernel(hs_hbm, src_hbm, dst_hbm, p0_hbm, p1_hbm, src_v, dst_v, rows_v, acc, sem):
        cid = lax.axis_index("c")
        sid = lax.axis_index("s")

        def zero_body(r, carry):
            for c in range(d // 16):
                rows_v[r, pl.ds(c * 16, 16)] = jnp.zeros((16,), jnp.float32)
            return carry

        lax.fori_loop(0, _CHUNK, zero_body, 0)
        zrow0 = sid * rows_per_tile
        for z in range(zc):
            pltpu.sync_copy(rows_v, acc.at[pl.ds(zrow0 + z * _CHUNK, _CHUNK)])
        plsc.subcore_barrier()

        def make_edge_body(start):
            def edge_body(g, carry):
                base = (start + g) * _CHUNK
                pltpu.sync_copy(src_hbm.at[pl.ds(base, _CHUNK)], src_v)
                pltpu.sync_copy(dst_hbm.at[pl.ds(base, _CHUNK)], dst_v)
                pltpu.async_copy(hs_hbm.at[src_v], rows_v, sem).wait()
                pltpu.sync_copy(rows_v, acc.at[dst_v], add=True)
                return carry

            return edge_body

        @pl.when(cid == 0)
        def _():
            lax.fori_loop(0, cpw0, make_edge_body(sid * cpw0), 0)

        @pl.when(cid == 1)
        def _():
            lax.fori_loop(0, cpw1, make_edge_body(_NS * cpw0 + sid * cpw1), 0)

        plsc.subcore_barrier()

        @pl.when(cid == 0)
        def _():
            pltpu.sync_copy(
                acc.at[pl.ds(zrow0, rows_per_tile)],
                p0_hbm.at[pl.ds(zrow0, rows_per_tile)],
            )

        @pl.when(cid == 1)
        def _():
            pltpu.sync_copy(
                acc.at[pl.ds(zrow0, rows_per_tile)],
                p1_hbm.at[pl.ds(zrow0, rows_per_tile)],
            )

    return agg_kernel


def _pick_bn(n):
    for bn in (1024, 1000, 512, 500, 256, 250, 128, 8):
        if n % bn == 0:
            return bn
    return n


def _tc_hs(x, w, d0, d1):
    n, d = x.shape
    bn = _pick_bn(n)

    def body(x_ref, w_ref, d0_ref, d1_ref, o_ref):
        deg = d0_ref[...][:, :1] + d1_ref[...][:, :1] + 1.0
        dinv = lax.rsqrt(deg)
        o_ref[...] = (
            jnp.dot(x_ref[...], w_ref[...], preferred_element_type=jnp.float32) * dinv
        )

    return pl.pallas_call(
        body,
        grid=(n // bn,),
        in_specs=[
            pl.BlockSpec((bn, d), lambda i: (i, 0)),
            pl.BlockSpec((d, d), lambda i: (0, 0)),
            pl.BlockSpec((bn, _DEGW), lambda i: (i, 0)),
            pl.BlockSpec((bn, _DEGW), lambda i: (i, 0)),
        ],
        out_specs=pl.BlockSpec((bn, d), lambda i: (i, 0)),
        out_shape=jax.ShapeDtypeStruct((n, d), jnp.float32),
    )(x, w, d0, d1)


def _tc_mid(p0, p1, hs1, d0, d1, b1, w2):
    n, d = hs1.shape
    bn = _pick_bn(n)

    def body(p0_ref, p1_ref, hs1_ref, d0_ref, d1_ref, b1_ref, w2_ref, t1_ref, hs2_ref):
        deg = d0_ref[...][:, :1] + d1_ref[...][:, :1] + 1.0
        dinv = lax.rsqrt(deg)
        t1 = jnp.tanh((p0_ref[...] + p1_ref[...] + hs1_ref[...]) * dinv + b1_ref[...])
        t1_ref[...] = t1
        hs2_ref[...] = (
            jnp.dot(t1, w2_ref[...], preferred_element_type=jnp.float32) * dinv
        )

    return pl.pallas_call(
        body,
        grid=(n // bn,),
        in_specs=[
            pl.BlockSpec((bn, d), lambda i: (i, 0)),
            pl.BlockSpec((bn, d), lambda i: (i, 0)),
            pl.BlockSpec((bn, d), lambda i: (i, 0)),
            pl.BlockSpec((bn, _DEGW), lambda i: (i, 0)),
            pl.BlockSpec((bn, _DEGW), lambda i: (i, 0)),
            pl.BlockSpec((1, d), lambda i: (0, 0)),
            pl.BlockSpec((d, d), lambda i: (0, 0)),
        ],
        out_specs=[
            pl.BlockSpec((bn, d), lambda i: (i, 0)),
            pl.BlockSpec((bn, d), lambda i: (i, 0)),
        ],
        out_shape=[
            jax.ShapeDtypeStruct((n, d), jnp.float32),
            jax.ShapeDtypeStruct((n, d), jnp.float32),
        ],
    )(p0, p1, hs1, d0, d1, b1, w2)


def _tc_fin(q0, q1, hs2, d0, d1, b2, t1):
    n, d = hs2.shape
    bn = _pick_bn(n)

    def body(q0_ref, q1_ref, hs2_ref, d0_ref, d1_ref, b2_ref, t1_ref, o_ref):
        deg = d0_ref[...][:, :1] + d1_ref[...][:, :1] + 1.0
        dinv = lax.rsqrt(deg)
        t2 = jnp.tanh((q0_ref[...] + q1_ref[...] + hs2_ref[...]) * dinv + b2_ref[...])
        o_ref[:, 0, :] = t1_ref[...]
        o_ref[:, 1, :] = t2

    return pl.pallas_call(
        body,
        grid=(n // bn,),
        in_specs=[
            pl.BlockSpec((bn, d), lambda i: (i, 0)),
            pl.BlockSpec((bn, d), lambda i: (i, 0)),
            pl.BlockSpec((bn, d), lambda i: (i, 0)),
            pl.BlockSpec((bn, _DEGW), lambda i: (i, 0)),
            pl.BlockSpec((bn, _DEGW), lambda i: (i, 0)),
            pl.BlockSpec((1, d), lambda i: (0, 0)),
            pl.BlockSpec((bn, d), lambda i: (i, 0)),
        ],
        out_specs=pl.BlockSpec((bn, 2, d), lambda i: (i, 0, 0)),
        out_shape=jax.ShapeDtypeStruct((n, 2, d), jnp.float32),
    )(q0, q1, hs2, d0, d1, b2, t1)


def kernel(x, edge_index, W1, b1, W2, b2):
    n, d = x.shape
    e = edge_index.shape[1]
    s_pair = 2 * _ceil_div(_ceil_div(e, _CHUNK), 2 * _NS)
    cpw0 = max(1, min(s_pair - 1, round(s_pair * _FRAC0)))
    cpw1 = s_pair - cpw0
    cpw = s_pair // 2
    e_pad = s_pair * _NS * _CHUNK
    pad = e_pad - e
    src = jnp.concatenate([edge_index[0], jnp.zeros((pad,), edge_index.dtype)])
    dst = jnp.concatenate([edge_index[1], jnp.full((pad,), n, edge_index.dtype)])
    n_acc = _ceil_div(n + 1, _NS * _CHUNK) * _NS * _CHUNK

    d0, d1 = _make_deg_kernel(n_acc, cpw)(dst)
    hs1 = _tc_hs(x, W1, d0, d1)
    agg = _make_agg_kernel(n, d, n_acc, cpw0, cpw1)
    p0, p1 = agg(hs1, src, dst)
    t1, hs2 = _tc_mid(p0, p1, hs1, d0, d1, b1.reshape(1, d), W2)
    q0, q1 = agg(hs2, src, dst)
    return _tc_fin(q0, q1, hs2, d0, d1, b2.reshape(1, d), t1)
